# baseline (speedup 1.0000x reference)
import jax
import jax.numpy as jnp
from jax.experimental import pallas as pl
from jax.experimental.pallas import tpu as pltpu

BATCH = 2048
NSAMP = 256
FEATP1 = 129
B_T = 64


def _dot(a, b):
    return jax.lax.dot_general(
        a, b, (((1,), (0,)), ((), ())), preferred_element_type=jnp.float32
    )


def _tanh_bf16(acc, b):
    return jnp.tanh(acc.astype(jnp.bfloat16) + b[...])


def _body(x_ref,
          ew0, eb0, ew1, eb1, ew2, eb2, ew3, eb3,
          dw0, db0, dw1, db1, dw2, db2, dw3, db3,
          out_ref):
    bt = B_T // 2

    def chain(a):
        acc = jax.lax.dot_general(a, ew0[...], (((0,), (0,)), ((), ())),
                                  preferred_element_type=jnp.float32)
        t = _tanh_bf16(acc, eb0)
        t = t.reshape(bt * 32, 1024)
        t = _tanh_bf16(_dot(t, ew1[...]), eb1)
        t = t.reshape(bt * 16, 256)
        t = _tanh_bf16(_dot(t, ew2[...]), eb2)
        t = t.reshape(bt * 2, 2048)
        t = _tanh_bf16(_dot(t, ew3[...]), eb3)
        t = _tanh_bf16(_dot(t, dw0[...]), db0)
        t = t.reshape(bt * 16, 256)
        t = _tanh_bf16(_dot(t, dw1[...]), db1)
        t = t.reshape(bt * 32, 128)
        t = _tanh_bf16(_dot(t, dw2[...]), db2)
        t = t.reshape(bt * 256, 128)
        acc = jax.lax.dot_general(dw3[...], t, (((0,), (1,)), ((), ())),
                                  preferred_element_type=jnp.float32)
        t = jnp.tanh(acc.astype(jnp.bfloat16) + db3[...])
        out = jnp.bfloat16(0.5) * t + jnp.bfloat16(0.5)
        return out.reshape(129, bt, 256).astype(jnp.float32)

    a = x_ref[...].astype(jnp.bfloat16)
    o1 = chain(a[:, :bt, :].reshape(129, bt * 256))
    o2 = chain(a[:, bt:, :].reshape(129, bt * 256))
    out_ref[...] = jnp.concatenate([o1, o2], axis=1)


@jax.jit
def kernel(x, enc_W0, enc_b0, enc_W1, enc_b1, enc_W2, enc_b2, enc_W3, enc_b3,
           dec_W0, dec_b0, dec_W1, dec_b1, dec_W2, dec_b2, dec_W3, dec_b3):
    xt = jnp.transpose(x, (2, 0, 1))

    raw = [(enc_W0, enc_b0), (enc_W1, enc_b1), (enc_W2, enc_b2),
           (enc_W3, enc_b3), (dec_W0, dec_b0), (dec_W1, dec_b1),
           (dec_W2, dec_b2), (dec_W3, dec_b3)]
    ws, bs = [], []
    for i, (w, b) in enumerate(raw):
        if i == 0:
            wf, bf = w * 0.5, b * 0.5
        else:
            wf = w * 0.25
            bf = jnp.sum(w, axis=0) * 0.25 + b * 0.5
        last = i == len(raw) - 1
        ws.append(wf.astype(jnp.bfloat16))
        if last:
            bs.append(bf.astype(jnp.bfloat16).reshape(-1, 1))
        else:
            bs.append(bf.astype(jnp.bfloat16).reshape(1, -1))

    def w_spec(w):
        return pl.BlockSpec(w.shape, lambda i: (0, 0))

    in_specs = [pl.BlockSpec((129, B_T, 256), lambda i: (0, i, 0))]
    operands = [xt]
    for w, b in zip(ws, bs):
        in_specs += [w_spec(w), w_spec(b)]
        operands += [w, b]

    out_t = pl.pallas_call(
        _body,
        grid=(BATCH // B_T,),
        in_specs=in_specs,
        out_specs=pl.BlockSpec((129, B_T, 256), lambda i: (0, i, 0)),
        out_shape=jax.ShapeDtypeStruct((FEATP1, BATCH, NSAMP), jnp.float32),
        compiler_params=pltpu.CompilerParams(
            dimension_semantics=("parallel",),
            vmem_limit_bytes=64 * 1024 * 1024,
        ),
    )(*operands)
    return jnp.transpose(out_t, (1, 2, 0))


# speedup vs baseline: 1.0747x; 1.0747x over previous
import jax
import jax.numpy as jnp
from jax.experimental import pallas as pl
from jax.experimental.pallas import tpu as pltpu

BATCH = 2048
NSAMP = 256
FEATP1 = 129
B_T = 64


def _dot(a, b):
    return jax.lax.dot_general(
        a, b, (((1,), (0,)), ((), ())), preferred_element_type=jnp.float32
    )


def _tanh_bf16(acc, b):
    return jnp.tanh(acc.astype(jnp.bfloat16) + b[...])


def _body(x_ref,
          ew0, eb0, ew1, eb1, ew2, eb2, ew3, eb3,
          dw0, db0, dw1, db1, dw2, db2, dw3, db3,
          out_ref):
    bt = B_T

    def chain(a):
        acc = jax.lax.dot_general(a, ew0[...], (((0,), (0,)), ((), ())),
                                  preferred_element_type=jnp.float32)
        t = _tanh_bf16(acc, eb0)
        t = t.reshape(bt * 32, 1024)
        t = _tanh_bf16(_dot(t, ew1[...]), eb1)
        t = t.reshape(bt * 16, 256)
        t = _tanh_bf16(_dot(t, ew2[...]), eb2)
        t = t.reshape(bt * 2, 2048)
        t = _tanh_bf16(_dot(t, ew3[...]), eb3)
        t = _tanh_bf16(_dot(t, dw0[...]), db0)
        t = t.reshape(bt * 16, 256)
        t = _tanh_bf16(_dot(t, dw1[...]), db1)
        t = t.reshape(bt * 32, 128)
        t = _tanh_bf16(_dot(t, dw2[...]), db2)
        t = t.reshape(bt * 256, 128)
        acc = jax.lax.dot_general(dw3[...], t, (((1,), (1,)), ((), ())),
                                  preferred_element_type=jnp.float32)
        t = jnp.tanh(acc.astype(jnp.bfloat16) + db3[...])
        out = jnp.bfloat16(0.5) * t + jnp.bfloat16(0.5)
        return out.reshape(129, bt, 256).astype(jnp.float32)

    a = x_ref[...].astype(jnp.bfloat16)
    out_ref[...] = chain(a.reshape(129, bt * 256))


@jax.jit
def kernel(x, enc_W0, enc_b0, enc_W1, enc_b1, enc_W2, enc_b2, enc_W3, enc_b3,
           dec_W0, dec_b0, dec_W1, dec_b1, dec_W2, dec_b2, dec_W3, dec_b3):
    xt = jnp.transpose(x, (2, 0, 1))

    raw = [(enc_W0, enc_b0), (enc_W1, enc_b1), (enc_W2, enc_b2),
           (enc_W3, enc_b3), (dec_W0, dec_b0), (dec_W1, dec_b1),
           (dec_W2, dec_b2), (dec_W3, dec_b3)]
    ws, bs = [], []
    for i, (w, b) in enumerate(raw):
        if i == 0:
            wf, bf = w * 0.5, b * 0.5
        else:
            wf = w * 0.25
            bf = jnp.sum(w, axis=0) * 0.25 + b * 0.5
        last = i == len(raw) - 1
        if last:
            wf = wf.T
        ws.append(wf.astype(jnp.bfloat16))
        if last:
            bs.append(bf.astype(jnp.bfloat16).reshape(-1, 1))
        else:
            bs.append(bf.astype(jnp.bfloat16).reshape(1, -1))

    def w_spec(w):
        return pl.BlockSpec(w.shape, lambda i: (0, 0))

    in_specs = [pl.BlockSpec((129, B_T, 256), lambda i: (0, i, 0))]
    operands = [xt]
    for w, b in zip(ws, bs):
        in_specs += [w_spec(w), w_spec(b)]
        operands += [w, b]

    out_t = pl.pallas_call(
        _body,
        grid=(BATCH // B_T,),
        in_specs=in_specs,
        out_specs=pl.BlockSpec((129, B_T, 256), lambda i: (0, i, 0)),
        out_shape=jax.ShapeDtypeStruct((FEATP1, BATCH, NSAMP), jnp.float32),
        compiler_params=pltpu.CompilerParams(
            dimension_semantics=("parallel",),
            vmem_limit_bytes=64 * 1024 * 1024,
        ),
    )(*operands)
    return jnp.transpose(out_t, (1, 2, 0))


# speedup vs baseline: 1.0755x; 1.0008x over previous
import jax
import jax.numpy as jnp
from jax.experimental import pallas as pl
from jax.experimental.pallas import tpu as pltpu

BATCH = 2048
NSAMP = 256
FEATP1 = 129
B_T = 64


def _dot(a, b):
    return jax.lax.dot_general(
        a, b, (((1,), (0,)), ((), ())), preferred_element_type=jnp.float32
    )


def _tanh_bf16(acc, b):
    return jnp.tanh(acc.astype(jnp.bfloat16) + b[...])


def _body(x_ref,
          ew0, eb0, ew1, eb1, ew2, eb2, ew3, eb3,
          dw0, db0, dw1, db1, dw2, db2, dw3, db3,
          out_ref):
    bt = B_T

    def chain(a):
        acc = jax.lax.dot_general(a, ew0[...], (((0,), (1,)), ((), ())),
                                  preferred_element_type=jnp.float32)
        t = _tanh_bf16(acc, eb0)
        t = t.reshape(bt * 32, 1024)
        t = _tanh_bf16(_dot(t, ew1[...]), eb1)
        t = t.reshape(bt * 16, 256)
        t = _tanh_bf16(_dot(t, ew2[...]), eb2)
        t = t.reshape(bt * 2, 2048)
        t = _tanh_bf16(_dot(t, ew3[...]), eb3)
        t = _tanh_bf16(_dot(t, dw0[...]), db0)
        t = t.reshape(bt * 16, 256)
        t = _tanh_bf16(_dot(t, dw1[...]), db1)
        t = t.reshape(bt * 32, 128)
        t = _tanh_bf16(_dot(t, dw2[...]), db2)
        t = t.reshape(bt * 256, 128)
        acc = jax.lax.dot_general(dw3[...], t, (((1,), (1,)), ((), ())),
                                  preferred_element_type=jnp.float32)
        t = jnp.tanh(acc.astype(jnp.bfloat16) + db3[...])
        out = jnp.bfloat16(0.5) * t + jnp.bfloat16(0.5)
        return out.reshape(129, bt, 256).astype(jnp.float32)

    a = x_ref[...].astype(jnp.bfloat16)
    out_ref[...] = chain(a.reshape(129, bt * 256))


@jax.jit
def kernel(x, enc_W0, enc_b0, enc_W1, enc_b1, enc_W2, enc_b2, enc_W3, enc_b3,
           dec_W0, dec_b0, dec_W1, dec_b1, dec_W2, dec_b2, dec_W3, dec_b3):
    xt = jnp.transpose(x, (2, 0, 1))

    raw = [(enc_W0, enc_b0), (enc_W1, enc_b1), (enc_W2, enc_b2),
           (enc_W3, enc_b3), (dec_W0, dec_b0), (dec_W1, dec_b1),
           (dec_W2, dec_b2), (dec_W3, dec_b3)]
    ws, bs = [], []
    for i, (w, b) in enumerate(raw):
        if i == 0:
            wf, bf = (w * 0.5).T, b * 0.5
        else:
            wf = w * 0.25
            bf = jnp.sum(w, axis=0) * 0.25 + b * 0.5
        last = i == len(raw) - 1
        if last:
            wf = wf.T
        ws.append(wf.astype(jnp.bfloat16))
        if last:
            bs.append(bf.astype(jnp.bfloat16).reshape(-1, 1))
        else:
            bs.append(bf.astype(jnp.bfloat16).reshape(1, -1))

    def w_spec(w):
        return pl.BlockSpec(w.shape, lambda i: (0, 0))

    in_specs = [pl.BlockSpec((129, B_T, 256), lambda i: (0, i, 0))]
    operands = [xt]
    for w, b in zip(ws, bs):
        in_specs += [w_spec(w), w_spec(b)]
        operands += [w, b]

    out_t = pl.pallas_call(
        _body,
        grid=(BATCH // B_T,),
        in_specs=in_specs,
        out_specs=pl.BlockSpec((129, B_T, 256), lambda i: (0, i, 0)),
        out_shape=jax.ShapeDtypeStruct((FEATP1, BATCH, NSAMP), jnp.float32),
        compiler_params=pltpu.CompilerParams(
            dimension_semantics=("parallel",),
            vmem_limit_bytes=64 * 1024 * 1024,
        ),
    )(*operands)
    return jnp.transpose(out_t, (1, 2, 0))
